# baseline (speedup 1.0000x reference)
import functools

import jax
import jax.numpy as jnp
from jax import lax
from jax.experimental import pallas as pl
from jax.experimental.pallas import tpu as pltpu
from jax.experimental.pallas import tpu_sc as plsc


def _build_gather(B, V, D):
    info = plsc.get_sparse_core_info()
    NC, NS, L = info.num_cores, info.num_subcores, info.num_lanes
    NW = NC * NS
    assert B % NW == 0 and D % L == 0
    b_per_w = B // NW
    mesh = plsc.VectorSubcoreMesh(core_axis_name="c", subcore_axis_name="s")

    @functools.partial(
        pl.kernel,
        mesh=mesh,
        out_type=jax.ShapeDtypeStruct((D, B), jnp.float32),
        scratch_types=[
            pltpu.VMEM((b_per_w,), jnp.int32),
            pltpu.VMEM((b_per_w, D), jnp.float32),
            pltpu.VMEM((D, b_per_w), jnp.float32),
            pltpu.SemaphoreType.DMA,
        ],
        compiler_params=pltpu.CompilerParams(
            use_tc_tiling_on_sc=False, needs_layout_passes=False
        ),
    )
    def gather_kernel(ids_hbm, table_hbm, out_t_hbm, idx_v, rows_v, rows_t_v, sem):
        wid = lax.axis_index("s") * NC + lax.axis_index("c")
        base = wid * b_per_w
        pltpu.sync_copy(ids_hbm.at[pl.ds(base, b_per_w)], idx_v)
        pltpu.async_copy(table_hbm.at[idx_v], rows_v, sem).wait()

        lane = lax.iota(jnp.int32, L)

        def transpose_row(i, carry):
            for h in range(D // L):
                part = rows_v[i, pl.ds(h * L, L)]
                plsc.store_scatter(
                    rows_t_v,
                    [lane + h * L, jnp.full((L,), i, jnp.int32)],
                    part,
                )
            return carry

        lax.fori_loop(0, b_per_w, transpose_row, 0, unroll=8)
        pltpu.sync_copy(rows_t_v, out_t_hbm.at[:, pl.ds(base, b_per_w)])

    return gather_kernel


def kernel(intent_ids, embedding_table):
    if intent_ids.ndim == 2:
        intent_ids = jnp.squeeze(intent_ids, axis=1)
    ids = intent_ids.astype(jnp.int32)
    B = ids.shape[0]
    V, D = embedding_table.shape
    one = lax.optimization_barrier(jnp.ones((), jnp.float32))
    table_rm = embedding_table * one
    out_t = _build_gather(B, V, D)(ids, table_rm)
    return out_t.T


# speedup vs baseline: 1.9688x; 1.9688x over previous
import functools

import jax
import jax.numpy as jnp
from jax import lax
from jax.experimental import pallas as pl
from jax.experimental.pallas import tpu as pltpu
from jax.experimental.pallas import tpu_sc as plsc


def _build_gather_t(B, V, D):
    info = plsc.get_sparse_core_info()
    NC, NS = info.num_cores, info.num_subcores
    NW = NC * NS
    assert B % NW == 0
    b_per_w = B // NW
    mesh = plsc.VectorSubcoreMesh(core_axis_name="c", subcore_axis_name="s")

    @functools.partial(
        pl.kernel,
        mesh=mesh,
        out_type=jax.ShapeDtypeStruct((D, B), jnp.float32),
        scratch_types=[
            pltpu.VMEM((b_per_w,), jnp.int32),
            pltpu.VMEM((D, b_per_w), jnp.float32),
            pltpu.SemaphoreType.DMA,
        ],
        compiler_params=pltpu.CompilerParams(use_tc_tiling_on_sc=False),
    )
    def gather_kernel(ids_hbm, table_t_hbm, out_t_hbm, idx_v, rows_v, sem):
        wid = lax.axis_index("s") * NC + lax.axis_index("c")
        base = wid * b_per_w
        pltpu.sync_copy(ids_hbm.at[pl.ds(base, b_per_w)], idx_v)
        gathers = [
            pltpu.async_copy(table_t_hbm.at[d].at[idx_v], rows_v.at[d], sem)
            for d in range(D)
        ]
        for g in gathers:
            g.wait()
        pltpu.sync_copy(rows_v, out_t_hbm.at[:, pl.ds(base, b_per_w)])

    return gather_kernel


def kernel(intent_ids, embedding_table):
    if intent_ids.ndim == 2:
        intent_ids = jnp.squeeze(intent_ids, axis=1)
    ids = intent_ids.astype(jnp.int32)
    B = ids.shape[0]
    V, D = embedding_table.shape
    out_t = _build_gather_t(B, V, D)(ids, embedding_table.T)
    return out_t.T
